# baseline (speedup 1.0000x reference)
import functools

import jax
import jax.numpy as jnp
from jax import lax
from jax.experimental import pallas as pl
from jax.experimental.pallas import tpu as pltpu
from jax.experimental.pallas import tpu_sc as plsc

VOCAB = 100000
HID = 128
B = 1024
S = 200
EPS = 1e-12

NW = 32
NTOK = B * S
TOK_PER_W = NTOK // NW
CHUNK = 128
NCHUNK = TOK_PER_W // CHUNK
NG = HID // 16
INV_HID = 1.0 / HID


def _rot_reduce(x, red, k):
    for shift in (8, 4, 2, 1):
        red[k, pl.ds(0, 16)] = x
        red[k, pl.ds(16, 16)] = x
        x = x + red[k, pl.ds(shift, 16)]
    return x


def _ln_rows(buf, ptab_v, p, gs, bs, red, k, r):
    xs = []
    for w in range(NG):
        xs.append(buf[r, pl.ds(16 * w, 16)] + ptab_v[p, pl.ds(16 * w, 16)])
    acc1 = xs[0]
    acc2 = xs[0] * xs[0]
    for w in range(1, NG):
        acc1 = acc1 + xs[w]
        acc2 = acc2 + xs[w] * xs[w]
    acc1 = _rot_reduce(acc1, red, 2 * k)
    acc2 = _rot_reduce(acc2, red, 2 * k + 1)
    mu = acc1 * INV_HID
    var = acc2 * INV_HID - mu * mu + EPS
    iv = lax.bitcast_convert_type(var, jnp.int32)
    y = lax.bitcast_convert_type(0x5F3759DF - (iv >> 1), jnp.float32)
    for _ in range(3):
        y = y * (1.5 - 0.5 * var * y * y)
    for w in range(NG):
        buf[r, pl.ds(16 * w, 16)] = (xs[w] - mu) * y * gs[w] + bs[w]


def _fused_kernel(ids_hbm, prow_hbm, table_hbm, ptab_hbm, gb_hbm, out_hbm,
                  idx_v, prow_v, buf0, buf1, ptab_v, gb_v, red, sem0, sem1):
    wid = lax.axis_index("s") * 2 + lax.axis_index("c")
    base = wid * TOK_PER_W
    pltpu.sync_copy(ids_hbm.at[wid], idx_v)
    pltpu.sync_copy(prow_hbm.at[wid], prow_v)
    pltpu.sync_copy(ptab_hbm, ptab_v)
    pltpu.sync_copy(gb_hbm, gb_v)
    gs = [gb_v[0, pl.ds(16 * w, 16)] for w in range(NG)]
    bs = [gb_v[1, pl.ds(16 * w, 16)] for w in range(NG)]
    bufs = (buf0, buf1)
    sems = (sem0, sem1)

    def start(c, b):
        off = pl.multiple_of(c * CHUNK, CHUNK)
        pltpu.async_copy(
            table_hbm.at[idx_v.at[pl.ds(off, CHUNK)]], bufs[b], sems[b])

    def drain(b):
        pltpu.make_async_copy(
            table_hbm.at[pl.ds(0, CHUNK)], bufs[b], sems[b]).wait()

    start(0, 0)
    start(1, 1)

    def outer(i, carry):
        for b in range(2):
            c = i * 2 + b
            cbase = pl.multiple_of(c * CHUNK, CHUNK)
            drain(b)

            def tokgroup(g, inner_carry):
                pvec = prow_v[pl.ds(cbase + 16 * g, 16)]
                for k in range(16):
                    _ln_rows(bufs[b], ptab_v, pvec[k], gs, bs, red,
                             k, 16 * g + k)
                return inner_carry

            lax.fori_loop(0, CHUNK // 16, tokgroup, 0)
            pltpu.sync_copy(bufs[b],
                            out_hbm.at[pl.ds(base + cbase, CHUNK)])

            @pl.when(c + 2 < NCHUNK)
            def _():
                start(c + 2, b)
        return carry

    lax.fori_loop(0, NCHUNK // 2, outer, 0)


def _sc_fused(ids, prow, word_emb, ptab, gb):
    mesh = plsc.VectorSubcoreMesh(core_axis_name="c", subcore_axis_name="s")
    k = functools.partial(
        pl.kernel,
        mesh=mesh,
        out_type=jax.ShapeDtypeStruct((NTOK, HID), jnp.float32),
        scratch_types=[
            pltpu.VMEM((TOK_PER_W,), jnp.int32),
            pltpu.VMEM((TOK_PER_W,), jnp.int32),
            pltpu.VMEM((CHUNK, HID), jnp.float32),
            pltpu.VMEM((CHUNK, HID), jnp.float32),
            pltpu.VMEM((2 * S, HID), jnp.float32),
            pltpu.VMEM((2, HID), jnp.float32),
            pltpu.VMEM((32, 32), jnp.float32),
            pltpu.SemaphoreType.DMA,
            pltpu.SemaphoreType.DMA,
        ],
    )(_fused_kernel)
    return k(ids, prow, word_emb, ptab, gb)


def kernel(input_ids, token_type_ids, word_emb, pos_emb, type_emb, gamma, beta):
    ids = input_ids.astype(jnp.int32).reshape(NW, TOK_PER_W)
    pos_ids = jnp.arange(S, dtype=jnp.int32)[None, :]
    prow = (token_type_ids.astype(jnp.int32) * S + pos_ids).reshape(NW, TOK_PER_W)
    ptab = (type_emb[:, None, :] + pos_emb[None, :S, :]).reshape(2 * S, HID)
    gb = jnp.stack([gamma, beta])
    rows = _sc_fused(ids, prow, word_emb, ptab, gb)
    return rows.reshape(B, S, HID)
